# baseline (speedup 1.0000x reference)
import functools

import jax
import jax.numpy as jnp
from jax import lax
from jax.experimental import pallas as pl
from jax.experimental.pallas import tpu as pltpu
from jax.experimental.pallas import tpu_sc as plsc


_TN = 2048
_TSUB = 1024


def _fold_tree(vals, idxs):
    while len(vals) > 1:
        nv, ni = [], []
        for p in range(0, len(vals), 2):
            vl, vr = vals[p], vals[p + 1]
            il, ir = idxs[p], idxs[p + 1]
            take_r = vr > vl
            nv.append(jnp.maximum(vl, vr))
            ni.append(jnp.where(take_r, ir, il))
        vals, idxs = nv, ni
    return vals[0], idxs[0]


def _argmax_body(x_ref, e_ref, idx_ref, en_s):
    c = e_ref.shape[0]

    @pl.when(pl.program_id(0) == 0)
    def _():
        e = e_ref[...]
        en_s[...] = e / jnp.clip(
            jnp.sqrt(jnp.sum(e * e, axis=-1, keepdims=True)), 1e-12, None)

    xb = x_ref[...]
    xn = xb / jnp.clip(
        jnp.sqrt(jnp.sum(xb * xb, axis=-1, keepdims=True)), 1e-12, None)

    pvals, pidxs = [], []
    for t in range(c // _TSUB):
        en = en_s[pl.ds(t * _TSUB, _TSUB), :]
        dist = lax.dot_general(
            xn, en, (((1,), (1,)), ((), ())),
            preferred_element_type=jnp.float32)
        vals = [dist[:, s * 128:(s + 1) * 128] for s in range(_TSUB // 128)]
        idxs = [jnp.full((_TN, 128), t * _TSUB + s * 128, jnp.int32)
                for s in range(_TSUB // 128)]
        v, ix = _fold_tree(vals, idxs)
        pvals.append(v)
        pidxs.append(ix)

    v, ix = _fold_tree(pvals, pidxs)
    rowmax = jnp.max(v, axis=-1)
    cidx = ix + lax.broadcasted_iota(jnp.int32, ix.shape, 1)
    cand = jnp.where(v == rowmax[:, None], cidx, jnp.int32(2**30))
    idx_ref[...] = jnp.min(cand, axis=-1).reshape(idx_ref.shape)


def _vq_argmax(x_flat, e, i0, nb):
    d = x_flat.shape[1]
    c = e.shape[0]
    out = pl.pallas_call(
        _argmax_body,
        grid=(nb,),
        in_specs=[
            pl.BlockSpec((_TN, d), lambda i: (i + i0, 0)),
            pl.BlockSpec((c, d), lambda i: (0, 0)),
        ],
        out_specs=pl.BlockSpec((_TN // 1024, 8, 128), lambda i: (i, 0, 0)),
        out_shape=jax.ShapeDtypeStruct((nb * (_TN // 1024), 8, 128), jnp.int32),
        scratch_shapes=[
            pltpu.VMEM((c, d), jnp.float32),
        ],
        compiler_params=pltpu.CompilerParams(
            dimension_semantics=("arbitrary",)),
    )(x_flat, e)
    return out


_NW = 32
_CH = 128


def _make_sc_gather(b, v, d):
    b_per_w = b // _NW
    n_ch = b_per_w // _CH
    mesh = plsc.VectorSubcoreMesh(core_axis_name="c", subcore_axis_name="s")

    @functools.partial(
        pl.kernel, mesh=mesh,
        out_type=jax.ShapeDtypeStruct((b, d), jnp.float32),
        scratch_types=[
            pltpu.VMEM((n_ch, _CH), jnp.int32),
            pltpu.VMEM((b_per_w, d), jnp.float32),
            pltpu.SemaphoreType.DMA,
        ],
        compiler_params=pltpu.CompilerParams(use_tc_tiling_on_sc=False),
    )
    def gather_k(idx_hbm, table_hbm, out_hbm, idx_v, rows_v, sem):
        wid = lax.axis_index("s") * 2 + lax.axis_index("c")
        base = wid * b_per_w
        tile = base // 1024
        row0 = (base % 1024) // _CH
        pltpu.sync_copy(idx_hbm.at[tile, pl.ds(row0, n_ch)], idx_v)
        copies = []
        for k in range(n_ch):
            copies.append(pltpu.async_copy(
                table_hbm.at[idx_v.at[k]],
                rows_v.at[pl.ds(k * _CH, _CH)], sem))
        for cp in copies:
            cp.wait()
        pltpu.sync_copy(rows_v, out_hbm.at[pl.ds(base, b_per_w)])

    return gather_k


def kernel(x, embed):
    xf = x.astype(jnp.float32)
    b0, b1, d = xf.shape
    c = embed.shape[1]
    n = b0 * b1

    x_flat = xf.reshape(n, d)
    e = embed.reshape(c, d)

    nb = n // _TN
    h = n // 2
    idx_t0 = _vq_argmax(x_flat, e, 0, nb // 2)
    idx_t1 = _vq_argmax(x_flat, e, nb // 2, nb // 2)
    gather = _make_sc_gather(h, c, d)
    q0 = gather(idx_t0, e)
    q1 = gather(idx_t1, e)
    quant = jnp.concatenate([q0, q1], axis=0)
    idx_tiles = jnp.concatenate([idx_t0, idx_t1], axis=0)

    return quant.reshape(b0, b1, d), idx_tiles.reshape(n).reshape(b0, b1)


# speedup vs baseline: 1.0608x; 1.0608x over previous
import functools

import jax
import jax.numpy as jnp
from jax import lax
from jax.experimental import pallas as pl
from jax.experimental.pallas import tpu as pltpu
from jax.experimental.pallas import tpu_sc as plsc


_TN = 2048
_TSUB = 1024


def _fold_tree(vals, idxs):
    while len(vals) > 1:
        nv, ni = [], []
        for p in range(0, len(vals), 2):
            vl, vr = vals[p], vals[p + 1]
            il, ir = idxs[p], idxs[p + 1]
            take_r = vr > vl
            nv.append(jnp.maximum(vl, vr))
            ni.append(jnp.where(take_r, ir, il))
        vals, idxs = nv, ni
    return vals[0], idxs[0]


def _argmax_body(x_ref, e_ref, idx_ref, en_s):
    c = e_ref.shape[0]

    @pl.when(pl.program_id(0) == 0)
    def _():
        e = e_ref[...]
        en_s[...] = e / jnp.clip(
            jnp.sqrt(jnp.sum(e * e, axis=-1, keepdims=True)), 1e-12, None)

    xb = x_ref[...]
    xn = xb / jnp.clip(
        jnp.sqrt(jnp.sum(xb * xb, axis=-1, keepdims=True)), 1e-12, None)

    pvals, pidxs = [], []
    for t in range(c // _TSUB):
        en = en_s[pl.ds(t * _TSUB, _TSUB), :]
        dist = lax.dot_general(
            xn, en, (((1,), (1,)), ((), ())),
            preferred_element_type=jnp.float32)
        vals = [dist[:, s * 128:(s + 1) * 128] for s in range(_TSUB // 128)]
        idxs = [jnp.full((_TN, 128), t * _TSUB + s * 128, jnp.int32)
                for s in range(_TSUB // 128)]
        v, ix = _fold_tree(vals, idxs)
        pvals.append(v)
        pidxs.append(ix)

    v, ix = _fold_tree(pvals, pidxs)
    rowmax = jnp.max(v, axis=-1)
    cidx = ix + lax.broadcasted_iota(jnp.int32, ix.shape, 1)
    cand = jnp.where(v == rowmax[:, None], cidx, jnp.int32(2**30))
    idx_ref[...] = jnp.min(cand, axis=-1).reshape(idx_ref.shape)


def _vq_argmax(x_flat, e):
    n, d = x_flat.shape
    c = e.shape[0]
    nb = n // _TN
    out = pl.pallas_call(
        _argmax_body,
        grid=(nb,),
        in_specs=[
            pl.BlockSpec((_TN, d), lambda i: (i, 0)),
            pl.BlockSpec((c, d), lambda i: (0, 0)),
        ],
        out_specs=pl.BlockSpec((_TN // 1024, 8, 128), lambda i: (i, 0, 0)),
        out_shape=jax.ShapeDtypeStruct((nb * (_TN // 1024), 8, 128), jnp.int32),
        scratch_shapes=[
            pltpu.VMEM((c, d), jnp.float32),
        ],
        compiler_params=pltpu.CompilerParams(
            dimension_semantics=("arbitrary",)),
    )(x_flat, e)
    return out


_NW = 32
_CH = 128


def _make_sc_gather(b, v, d):
    b_per_w = b // _NW
    n_ch = b_per_w // _CH
    mesh = plsc.VectorSubcoreMesh(core_axis_name="c", subcore_axis_name="s")

    @functools.partial(
        pl.kernel, mesh=mesh,
        out_type=jax.ShapeDtypeStruct((b, d), jnp.float32),
        scratch_types=[
            pltpu.VMEM((n_ch, _CH), jnp.int32),
            pltpu.VMEM((b_per_w, d), jnp.float32),
            pltpu.SemaphoreType.DMA,
        ],
        compiler_params=pltpu.CompilerParams(use_tc_tiling_on_sc=False),
    )
    def gather_k(idx_hbm, table_hbm, out_hbm, idx_v, rows_v, sem):
        wid = lax.axis_index("s") * 2 + lax.axis_index("c")
        base = wid * b_per_w
        pltpu.sync_copy(idx_hbm.at[wid], idx_v)
        copies = []
        for k in range(n_ch):
            copies.append(pltpu.async_copy(
                table_hbm.at[idx_v.at[k]],
                rows_v.at[pl.ds(k * _CH, _CH)], sem))
        for cp in copies:
            cp.wait()
        pltpu.sync_copy(rows_v, out_hbm.at[pl.ds(base, b_per_w)])

    return gather_k


def kernel(x, embed):
    xf = x.astype(jnp.float32)
    b0, b1, d = xf.shape
    c = embed.shape[1]
    n = b0 * b1

    x_flat = xf.reshape(n, d)
    e = embed.reshape(c, d)

    idx_tiles = _vq_argmax(x_flat, e)

    quant = _make_sc_gather(n, c, d)(idx_tiles, e)

    return quant.reshape(b0, b1, d), idx_tiles.reshape(n).reshape(b0, b1)
